# baseline (speedup 1.0000x reference)
import functools

import jax
import jax.numpy as jnp
from jax import lax
from jax.experimental import pallas as pl
from jax.experimental.pallas import tpu as pltpu
from jax.experimental.pallas import tpu_sc as plsc

_PACK_BR = 4000

_NC = 2
_NS = 16
_NW = _NC * _NS

_GW = 128


def _sc_gather(o_idx, s_idx, r_idx, E2, R2, *, B, NEG, D2):
    BW = B // _NW
    CH = (BW * NEG) // _GW

    mesh = plsc.VectorSubcoreMesh(core_axis_name="c", subcore_axis_name="s")

    K = 2
    SLAB = K * _GW
    NSLAB = CH // K

    @functools.partial(
        pl.kernel,
        out_type=[
            jax.ShapeDtypeStruct((B * NEG, D2), jnp.float32),
            jax.ShapeDtypeStruct((B, D2), jnp.float32),
            jax.ShapeDtypeStruct((B, D2), jnp.float32),
        ],
        mesh=mesh,
        scratch_types=[
            pltpu.VMEM((CH, _GW), jnp.int32),
            pltpu.VMEM((BW,), jnp.int32),
            pltpu.VMEM((BW,), jnp.int32),
            pltpu.VMEM((SLAB, D2), jnp.float32),
            pltpu.VMEM((SLAB, D2), jnp.float32),
            pltpu.VMEM((BW, D2), jnp.float32),
            pltpu.VMEM((BW, D2), jnp.float32),
            pltpu.SemaphoreType.DMA,
            pltpu.SemaphoreType.DMA,
            pltpu.SemaphoreType.DMA,
            pltpu.SemaphoreType.DMA,
            pltpu.SemaphoreType.DMA,
        ],
    )
    def sc_kernel(oidx_hbm, sidx_hbm, ridx_hbm, E_hbm, R_hbm,
                  oE_hbm, sE_hbm, rE_hbm,
                  oidx_v, sidx_v, ridx_v, slab0, slab1, srows_v, rrows_v,
                  semg0, semg1, semw0, semw1, semsr):
        wid = lax.axis_index("s") * _NC + lax.axis_index("c")
        base = wid * BW * NEG

        pltpu.sync_copy(sidx_hbm.at[wid], sidx_v)
        pltpu.sync_copy(ridx_hbm.at[wid], ridx_v)
        pltpu.sync_copy(oidx_hbm.at[wid], oidx_v)

        pltpu.async_copy(E_hbm.at[sidx_v], srows_v, semsr)
        pltpu.async_copy(R_hbm.at[ridx_v], rrows_v, semsr)

        slabs = (slab0, slab1)
        gsems = (semg0, semg1)
        wsems = (semw0, semw1)

        def fire(p, step):
            for t in range(K):
                pltpu.async_copy(
                    E_hbm.at[oidx_v.at[step * K + t]],
                    slabs[p].at[pl.ds(t * _GW, _GW)], gsems[p])

        def drain_gathers(p):
            for t in range(K):
                pltpu.make_async_copy(
                    E_hbm.at[oidx_v.at[t]],
                    slabs[p].at[pl.ds(t * _GW, _GW)], gsems[p]).wait()

        def start_wb(p, step):
            pltpu.async_copy(
                slabs[p], oE_hbm.at[pl.ds(base + step * SLAB, SLAB)], wsems[p])

        def drain_wb(p):
            pltpu.make_async_copy(
                slabs[p], oE_hbm.at[pl.ds(base, SLAB)], wsems[p]).wait()

        fire(0, 0)

        @pl.loop(0, NSLAB, step=2)
        def _(n):
            @pl.when(n > 0)
            def _():
                drain_wb(1)
            fire(1, n + 1)
            drain_gathers(0)
            start_wb(0, n)
            drain_gathers(1)
            start_wb(1, n + 1)

            @pl.when(n + 2 < NSLAB)
            def _():
                drain_wb(0)
                fire(0, n + 2)

        drain_wb(0)
        drain_wb(1)

        pltpu.make_async_copy(E_hbm.at[sidx_v], srows_v, semsr).wait()
        pltpu.make_async_copy(R_hbm.at[ridx_v], rrows_v, semsr).wait()
        pltpu.sync_copy(srows_v, sE_hbm.at[pl.ds(wid * BW, BW)])
        pltpu.sync_copy(rrows_v, rE_hbm.at[pl.ds(wid * BW, BW)])

    return sc_kernel(o_idx, s_idx, r_idx, E2, R2)


def _tc_pack(E, *, D):
    N = E.shape[0]
    H = N // 2
    BR = _PACK_BR
    CH8 = 2 * BR
    nblk = N // CH8

    ETr = E.T.reshape(D, N // 1000, 1000)

    def body(x_ref, o_ref):
        x = x_ref[...]
        for a in range(4):
            o_ref[pl.ds(1000 * a, 1000), :D] = x[:, a, :].T
            o_ref[pl.ds(1000 * a, 1000), D:] = x[:, a + 4, :].T

    return pl.pallas_call(
        body,
        grid=(nblk,),
        in_specs=[pl.BlockSpec((D, 8, 1000), lambda i: (0, i, 0))],
        out_specs=pl.BlockSpec((BR, 2 * D), lambda i: (i, 0)),
        out_shape=jax.ShapeDtypeStruct((H, 2 * D), jnp.float32),
    )(ETr)


def _tc_reduce(sE, rE, oE3, sp, rp, op3, *, B, NEG, D, D2):
    BB = 128

    def body(s_ref, r_ref, sp_ref, rp_ref, op_ref, o_ref, out_ref):
        s_pair = s_ref[...]
        r_pair = r_ref[...]
        s_half = jnp.where(sp_ref[...] == 1, s_pair[:, D:], s_pair[:, :D])
        r_half = jnp.where(rp_ref[...] == 1, r_pair[:, D:], r_pair[:, :D])
        q = s_half * r_half
        q2 = jnp.concatenate([q, q], axis=-1)
        prod = o_ref[...] * q2[:, None, :]
        lane = lax.broadcasted_iota(jnp.int32, (BB, NEG, D2), 2)
        keep = (lane >= D) == (op_ref[...][:, :, None] == 1)
        out_ref[...] = jnp.sum(jnp.where(keep, prod, 0.0), axis=-1)

    return pl.pallas_call(
        body,
        grid=(B // BB,),
        in_specs=[
            pl.BlockSpec((BB, D2), lambda i: (i, 0)),
            pl.BlockSpec((BB, D2), lambda i: (i, 0)),
            pl.BlockSpec((BB, 1), lambda i: (i, 0)),
            pl.BlockSpec((BB, 1), lambda i: (i, 0)),
            pl.BlockSpec((BB, NEG), lambda i: (i, 0)),
            pl.BlockSpec((BB, NEG, D2), lambda i: (i, 0, 0)),
        ],
        out_specs=pl.BlockSpec((BB, NEG), lambda i: (i, 0)),
        out_shape=jax.ShapeDtypeStruct((B, NEG), jnp.float32),
    )(sE, rE, sp, rp, op3, oE3)


def kernel(s, r, o, E, R):
    B, NEG = o.shape
    D = E.shape[1]
    D2 = 2 * D

    E2 = _tc_pack(E, D=D)
    R2 = R.reshape(R.shape[0] // 2, D2)

    H = E.shape[0] // 2
    o32 = o.astype(jnp.int32)
    s32 = s.astype(jnp.int32).reshape(B, 1)
    r32 = r.astype(jnp.int32).reshape(B, 1)

    BR = _PACK_BR
    o_g, o_j = o32 // (2 * BR), o32 % (2 * BR)
    s_g, s_j = s32 // (2 * BR), s32 % (2 * BR)
    o_par = (o_j >= BR).astype(jnp.int32)
    s_par = (s_j >= BR).astype(jnp.int32)
    o_idx = (o_g * BR + o_j - BR * o_par).reshape(
        _NW, (B * NEG) // (_NW * _GW), _GW)
    s_idx = (s_g * BR + s_j - BR * s_par).reshape(_NW, B // _NW)
    r_idx = (r32 >> 1).reshape(_NW, B // _NW)

    oE, sE, rE = _sc_gather(o_idx, s_idx, r_idx, E2, R2, B=B, NEG=NEG, D2=D2)
    return _tc_reduce(sE, rE, oE.reshape(B, NEG, D2),
                      s_par, r32 & 1, o_par, B=B, NEG=NEG, D=D, D2=D2)


# speedup vs baseline: 1.4159x; 1.4159x over previous
import functools

import jax
import jax.numpy as jnp
from jax import lax
from jax.experimental import pallas as pl
from jax.experimental.pallas import tpu as pltpu
from jax.experimental.pallas import tpu_sc as plsc

_PACK_BR = 4000

_NC = 2
_NS = 16
_NW = _NC * _NS

_GW = 128


def _sc_gather(o_idx, s_idx, r_idx, E2, R2, *, B, NEG, D2):
    BW = B // _NW
    CH = (BW * NEG) // _GW

    mesh = plsc.VectorSubcoreMesh(core_axis_name="c", subcore_axis_name="s")

    K = 2
    SLAB = K * _GW
    NSLAB = CH // K

    @functools.partial(
        pl.kernel,
        out_type=[
            jax.ShapeDtypeStruct((B * NEG, D2), jnp.float32),
            jax.ShapeDtypeStruct((B, D2), jnp.float32),
            jax.ShapeDtypeStruct((B, D2), jnp.float32),
        ],
        mesh=mesh,
        scratch_types=[
            pltpu.VMEM((CH, _GW), jnp.int32),
            pltpu.VMEM((BW,), jnp.int32),
            pltpu.VMEM((BW,), jnp.int32),
            pltpu.VMEM((SLAB, D2), jnp.float32),
            pltpu.VMEM((SLAB, D2), jnp.float32),
            pltpu.VMEM((BW, D2), jnp.float32),
            pltpu.VMEM((BW, D2), jnp.float32),
            pltpu.SemaphoreType.DMA,
            pltpu.SemaphoreType.DMA,
            pltpu.SemaphoreType.DMA,
            pltpu.SemaphoreType.DMA,
            pltpu.SemaphoreType.DMA,
        ],
    )
    def sc_kernel(oidx_hbm, sidx_hbm, ridx_hbm, E_hbm, R_hbm,
                  oE_hbm, sE_hbm, rE_hbm,
                  oidx_v, sidx_v, ridx_v, slab0, slab1, srows_v, rrows_v,
                  semg0, semg1, semw0, semw1, semsr):
        wid = lax.axis_index("s") * _NC + lax.axis_index("c")
        base = wid * BW * NEG

        pltpu.sync_copy(sidx_hbm.at[wid], sidx_v)
        pltpu.sync_copy(ridx_hbm.at[wid], ridx_v)
        pltpu.sync_copy(oidx_hbm.at[wid], oidx_v)

        pltpu.async_copy(E_hbm.at[sidx_v], srows_v, semsr)
        pltpu.async_copy(R_hbm.at[ridx_v], rrows_v, semsr)

        slabs = (slab0, slab1)
        gsems = (semg0, semg1)
        wsems = (semw0, semw1)

        def fire(p, step):
            for t in range(K):
                pltpu.async_copy(
                    E_hbm.at[oidx_v.at[step * K + t]],
                    slabs[p].at[pl.ds(t * _GW, _GW)], gsems[p])

        def drain_gathers(p):
            for t in range(K):
                pltpu.make_async_copy(
                    E_hbm.at[oidx_v.at[t]],
                    slabs[p].at[pl.ds(t * _GW, _GW)], gsems[p]).wait()

        def start_wb(p, step):
            pltpu.async_copy(
                slabs[p], oE_hbm.at[pl.ds(base + step * SLAB, SLAB)], wsems[p])

        def drain_wb(p):
            pltpu.make_async_copy(
                slabs[p], oE_hbm.at[pl.ds(base, SLAB)], wsems[p]).wait()

        fire(0, 0)

        @pl.loop(0, NSLAB, step=2)
        def _(n):
            @pl.when(n > 0)
            def _():
                drain_wb(1)
            fire(1, n + 1)
            drain_gathers(0)
            start_wb(0, n)
            drain_gathers(1)
            start_wb(1, n + 1)

            @pl.when(n + 2 < NSLAB)
            def _():
                drain_wb(0)
                fire(0, n + 2)

        drain_wb(0)
        drain_wb(1)

        pltpu.make_async_copy(E_hbm.at[sidx_v], srows_v, semsr).wait()
        pltpu.make_async_copy(R_hbm.at[ridx_v], rrows_v, semsr).wait()
        pltpu.sync_copy(srows_v, sE_hbm.at[pl.ds(wid * BW, BW)])
        pltpu.sync_copy(rrows_v, rE_hbm.at[pl.ds(wid * BW, BW)])

    return sc_kernel(o_idx, s_idx, r_idx, E2, R2)


def _tc_pack(E, *, D):
    N = E.shape[0]
    H = N // 2
    BR = _PACK_BR
    nblk = H // BR
    G = 2 * BR // 16

    E3 = E.reshape(N // 16, 16, D)

    def body(x_ref, o_ref):
        x = x_ref[...].reshape(2 * BR, D)
        o_ref[:, :D] = x[:BR, :]
        o_ref[:, D:] = x[BR:, :]

    return pl.pallas_call(
        body,
        grid=(nblk,),
        in_specs=[pl.BlockSpec((G, 16, D), lambda i: (i, 0, 0))],
        out_specs=pl.BlockSpec((BR, 2 * D), lambda i: (i, 0)),
        out_shape=jax.ShapeDtypeStruct((H, 2 * D), jnp.float32),
    )(E3)


def _tc_reduce(sE, rE, oE3, sp, rp, op3, *, B, NEG, D, D2):
    BB = 256

    def body(s_ref, r_ref, sp_ref, rp_ref, op_ref, o_ref, out_ref):
        s_pair = s_ref[...]
        r_pair = r_ref[...]
        s_half = jnp.where(sp_ref[...] == 1, s_pair[:, D:], s_pair[:, :D])
        r_half = jnp.where(rp_ref[...] == 1, r_pair[:, D:], r_pair[:, :D])
        q = s_half * r_half
        q2 = jnp.concatenate([q, q], axis=-1)
        prod = o_ref[...] * q2[:, None, :]
        sum_l = jnp.sum(prod[..., :D], axis=-1)
        sum_r = jnp.sum(prod[..., D:], axis=-1)
        out_ref[...] = jnp.where(op_ref[...] == 1, sum_r, sum_l)

    return pl.pallas_call(
        body,
        grid=(B // BB,),
        in_specs=[
            pl.BlockSpec((BB, D2), lambda i: (i, 0)),
            pl.BlockSpec((BB, D2), lambda i: (i, 0)),
            pl.BlockSpec((BB, 1), lambda i: (i, 0)),
            pl.BlockSpec((BB, 1), lambda i: (i, 0)),
            pl.BlockSpec((BB, NEG), lambda i: (i, 0)),
            pl.BlockSpec((BB, NEG, D2), lambda i: (i, 0, 0)),
        ],
        out_specs=pl.BlockSpec((BB, NEG), lambda i: (i, 0)),
        out_shape=jax.ShapeDtypeStruct((B, NEG), jnp.float32),
    )(sE, rE, sp, rp, op3, oE3)


def kernel(s, r, o, E, R):
    B, NEG = o.shape
    D = E.shape[1]
    D2 = 2 * D

    E2 = _tc_pack(E, D=D)
    R2 = R.reshape(R.shape[0] // 2, D2)

    H = E.shape[0] // 2
    o32 = o.astype(jnp.int32)
    s32 = s.astype(jnp.int32).reshape(B, 1)
    r32 = r.astype(jnp.int32).reshape(B, 1)

    BR = _PACK_BR
    o_g, o_j = o32 // (2 * BR), o32 % (2 * BR)
    s_g, s_j = s32 // (2 * BR), s32 % (2 * BR)
    o_par = (o_j >= BR).astype(jnp.int32)
    s_par = (s_j >= BR).astype(jnp.int32)
    o_idx = (o_g * BR + o_j - BR * o_par).reshape(
        _NW, (B * NEG) // (_NW * _GW), _GW)
    s_idx = (s_g * BR + s_j - BR * s_par).reshape(_NW, B // _NW)
    r_idx = (r32 >> 1).reshape(_NW, B // _NW)

    oE, sE, rE = _sc_gather(o_idx, s_idx, r_idx, E2, R2, B=B, NEG=NEG, D2=D2)
    return _tc_reduce(sE, rE, oE.reshape(B, NEG, D2),
                      s_par, r32 & 1, o_par, B=B, NEG=NEG, D=D, D2=D2)


# speedup vs baseline: 1.4865x; 1.0499x over previous
import functools

import jax
import jax.numpy as jnp
from jax import lax
from jax.experimental import pallas as pl
from jax.experimental.pallas import tpu as pltpu
from jax.experimental.pallas import tpu_sc as plsc

_PACK_BR = 4000

_NC = 2
_NS = 16
_NW = _NC * _NS

_GW = 128


def _sc_gather(o_idx, s_idx, r_idx, E2, R2, *, B, NEG, D2):
    BW = B // _NW
    CH = (BW * NEG) // _GW

    mesh = plsc.VectorSubcoreMesh(core_axis_name="c", subcore_axis_name="s")

    K = 2
    SLAB = K * _GW
    NSLAB = CH // K

    @functools.partial(
        pl.kernel,
        out_type=[
            jax.ShapeDtypeStruct((B * NEG, D2), jnp.float32),
            jax.ShapeDtypeStruct((B, D2), jnp.float32),
            jax.ShapeDtypeStruct((B, D2), jnp.float32),
        ],
        mesh=mesh,
        scratch_types=[
            pltpu.VMEM((CH, _GW), jnp.int32),
            pltpu.VMEM((BW,), jnp.int32),
            pltpu.VMEM((BW,), jnp.int32),
            pltpu.VMEM((SLAB, D2), jnp.float32),
            pltpu.VMEM((SLAB, D2), jnp.float32),
            pltpu.VMEM((BW, D2), jnp.float32),
            pltpu.VMEM((BW, D2), jnp.float32),
            pltpu.SemaphoreType.DMA,
            pltpu.SemaphoreType.DMA,
            pltpu.SemaphoreType.DMA,
            pltpu.SemaphoreType.DMA,
            pltpu.SemaphoreType.DMA,
        ],
    )
    def sc_kernel(oidx_hbm, sidx_hbm, ridx_hbm, E_hbm, R_hbm,
                  oE_hbm, sE_hbm, rE_hbm,
                  oidx_v, sidx_v, ridx_v, slab0, slab1, srows_v, rrows_v,
                  semg0, semg1, semw0, semw1, semsr):
        wid = lax.axis_index("s") * _NC + lax.axis_index("c")
        base = wid * BW * NEG

        pltpu.sync_copy(sidx_hbm.at[wid], sidx_v)
        pltpu.sync_copy(ridx_hbm.at[wid], ridx_v)
        pltpu.sync_copy(oidx_hbm.at[wid], oidx_v)

        pltpu.async_copy(E_hbm.at[sidx_v], srows_v, semsr)
        pltpu.async_copy(R_hbm.at[ridx_v], rrows_v, semsr)

        slabs = (slab0, slab1)
        gsems = (semg0, semg1)
        wsems = (semw0, semw1)

        def fire(p, step):
            for t in range(K):
                pltpu.async_copy(
                    E_hbm.at[oidx_v.at[step * K + t]],
                    slabs[p].at[pl.ds(t * _GW, _GW)], gsems[p])

        def drain_gathers(p):
            for t in range(K):
                pltpu.make_async_copy(
                    E_hbm.at[oidx_v.at[t]],
                    slabs[p].at[pl.ds(t * _GW, _GW)], gsems[p]).wait()

        def start_wb(p, step):
            pltpu.async_copy(
                slabs[p], oE_hbm.at[pl.ds(base + step * SLAB, SLAB)], wsems[p])

        def drain_wb(p):
            pltpu.make_async_copy(
                slabs[p], oE_hbm.at[pl.ds(base, SLAB)], wsems[p]).wait()

        fire(0, 0)

        @pl.loop(0, NSLAB, step=2)
        def _(n):
            @pl.when(n > 0)
            def _():
                drain_wb(1)
            fire(1, n + 1)
            drain_gathers(0)
            start_wb(0, n)
            drain_gathers(1)
            start_wb(1, n + 1)

            @pl.when(n + 2 < NSLAB)
            def _():
                drain_wb(0)
                fire(0, n + 2)

        drain_wb(0)
        drain_wb(1)

        pltpu.make_async_copy(E_hbm.at[sidx_v], srows_v, semsr).wait()
        pltpu.make_async_copy(R_hbm.at[ridx_v], rrows_v, semsr).wait()
        pltpu.sync_copy(srows_v, sE_hbm.at[pl.ds(wid * BW, BW)])
        pltpu.sync_copy(rrows_v, rE_hbm.at[pl.ds(wid * BW, BW)])

    return sc_kernel(o_idx, s_idx, r_idx, E2, R2)


def _tc_pack(E, *, D):
    N = E.shape[0]
    H = N // 2
    BR = _PACK_BR
    nblk = H // BR
    G = 2 * BR // 16

    E3 = E.reshape(N // 16, 16, D)

    def body(x_ref, o_ref):
        x = x_ref[...].reshape(2 * BR, D)
        o_ref[:, :D] = x[:BR, :]
        o_ref[:, D:] = x[BR:, :]

    return pl.pallas_call(
        body,
        grid=(nblk,),
        in_specs=[pl.BlockSpec((G, 16, D), lambda i: (i, 0, 0))],
        out_specs=pl.BlockSpec((BR, 2 * D), lambda i: (i, 0)),
        out_shape=jax.ShapeDtypeStruct((H, 2 * D), jnp.float32),
    )(E3)


def _tc_reduce(sE, rE, oE3, sp, rp, op3, *, B, NEG, D, D2):
    BB = 128

    def body(s_ref, r_ref, sp_ref, rp_ref, op_ref, o_ref, out_ref):
        s_pair = s_ref[...]
        r_pair = r_ref[...]
        s_half = jnp.where(sp_ref[...] == 1, s_pair[:, D:], s_pair[:, :D])
        r_half = jnp.where(rp_ref[...] == 1, r_pair[:, D:], r_pair[:, :D])
        q = s_half * r_half
        q2 = jnp.concatenate([q, q], axis=-1)
        prod = o_ref[...] * q2[:, None, :]
        lane = lax.broadcasted_iota(jnp.int32, (BB, NEG, D2), 2)
        keep = (lane >= D) == (op_ref[...][:, :, None] == 1)
        out_ref[...] = jnp.sum(jnp.where(keep, prod, 0.0), axis=-1)

    return pl.pallas_call(
        body,
        grid=(B // BB,),
        in_specs=[
            pl.BlockSpec((BB, D2), lambda i: (i, 0)),
            pl.BlockSpec((BB, D2), lambda i: (i, 0)),
            pl.BlockSpec((BB, 1), lambda i: (i, 0)),
            pl.BlockSpec((BB, 1), lambda i: (i, 0)),
            pl.BlockSpec((BB, NEG), lambda i: (i, 0)),
            pl.BlockSpec((BB, NEG, D2), lambda i: (i, 0, 0)),
        ],
        out_specs=pl.BlockSpec((BB, NEG), lambda i: (i, 0)),
        out_shape=jax.ShapeDtypeStruct((B, NEG), jnp.float32),
    )(sE, rE, sp, rp, op3, oE3)


def kernel(s, r, o, E, R):
    B, NEG = o.shape
    D = E.shape[1]
    D2 = 2 * D

    E2 = _tc_pack(E, D=D)
    R2 = R.reshape(R.shape[0] // 2, D2)

    H = E.shape[0] // 2
    o32 = o.astype(jnp.int32)
    s32 = s.astype(jnp.int32).reshape(B, 1)
    r32 = r.astype(jnp.int32).reshape(B, 1)

    BR = _PACK_BR
    o_g, o_j = o32 // (2 * BR), o32 % (2 * BR)
    s_g, s_j = s32 // (2 * BR), s32 % (2 * BR)
    o_par = (o_j >= BR).astype(jnp.int32)
    s_par = (s_j >= BR).astype(jnp.int32)
    o_idx = (o_g * BR + o_j - BR * o_par).reshape(
        _NW, (B * NEG) // (_NW * _GW), _GW)
    s_idx = (s_g * BR + s_j - BR * s_par).reshape(_NW, B // _NW)
    r_idx = (r32 >> 1).reshape(_NW, B // _NW)

    oE, sE, rE = _sc_gather(o_idx, s_idx, r_idx, E2, R2, B=B, NEG=NEG, D2=D2)
    return _tc_reduce(sE, rE, oE.reshape(B, NEG, D2),
                      s_par, r32 & 1, o_par, B=B, NEG=NEG, D=D, D2=D2)
